# baseline (speedup 1.0000x reference)
import functools

import jax
import jax.numpy as jnp
from jax import lax
from jax.experimental import pallas as pl
from jax.experimental.pallas import tpu as pltpu
from jax.experimental.pallas import tpu_sc as plsc

D = 64
DP = 128


def _make_gather(B: int, H: int, V: int):
    info = plsc.get_sparse_core_info()
    NC, NS, L = info.num_cores, info.num_subcores, info.num_lanes
    NW = NC * NS
    assert B % NW == 0 and H % 8 == 0
    rows_per_w = B // NW
    mesh = plsc.VectorSubcoreMesh(core_axis_name="c", subcore_axis_name="s")

    @functools.partial(
        pl.kernel,
        mesh=mesh,
        out_type=jax.ShapeDtypeStruct((B, H, D), jnp.float32),
        scratch_types=[
            pltpu.VMEM((rows_per_w // 2, H), jnp.int32),
            pltpu.VMEM((2, H, DP), jnp.float32),
            pltpu.VMEM((2, H, D), jnp.float32),
            pltpu.SemaphoreType.DMA,
            pltpu.SemaphoreType.DMA,
            pltpu.SemaphoreType.DMA,
            pltpu.SemaphoreType.DMA,
        ],
        compiler_params=pltpu.CompilerParams(use_tc_tiling_on_sc=True),
    )
    def gather_kernel(x_hbm, t128_hbm, out_hbm, idx2d, rows_v, s64_v,
                      gsem, csem, ssem0, ssem1):
        wid = lax.axis_index("s") * NC + lax.axis_index("c")
        base = wid * rows_per_w
        half = rows_per_w // 2

        def stage_idx(h):
            pltpu.sync_copy(x_hbm.at[pl.ds(base + h * half, half)], idx2d)

        stage_idx(0)

        ssems = (ssem0, ssem1)

        def run_row(b_loc, b_glob, buf, wait_store):
            store_src = s64_v.at[buf]
            store_dst = out_hbm.at[base + b_glob]
            if wait_store:
                pltpu.make_async_copy(store_src, store_dst, ssems[buf]).wait()
            d1 = pltpu.async_copy(
                t128_hbm.at[idx2d.at[b_loc, pl.ds(0, 128)]],
                rows_v.at[buf].at[pl.ds(0, 128)], gsem)
            d2 = pltpu.async_copy(
                t128_hbm.at[idx2d.at[b_loc, pl.ds(128, H - 128)]],
                rows_v.at[buf].at[pl.ds(128, H - 128)], gsem)
            d1.wait()
            d2.wait()

            def compact(h, carry):
                for v in range(D // L):
                    s64_v[buf, h, pl.ds(v * L, L)] = (
                        rows_v[buf, h, pl.ds(v * L, L)])
                return carry

            lax.fori_loop(0, H, compact, 0, unroll=8)
            pltpu.async_copy(store_src, store_dst, ssems[buf])

        run_row(0, 0, 0, False)
        run_row(1, 1, 1, False)

        def body_a(p, carry):
            run_row(2 * p, 2 * p, 0, True)
            run_row(2 * p + 1, 2 * p + 1, 1, True)
            return carry

        lax.fori_loop(1, half // 2, body_a, 0, unroll=False)

        stage_idx(1)

        def body_b(p, carry):
            run_row(2 * p, half + 2 * p, 0, True)
            run_row(2 * p + 1, half + 2 * p + 1, 1, True)
            return carry

        lax.fori_loop(0, half // 2, body_b, 0, unroll=False)

        pltpu.make_async_copy(
            s64_v.at[0], out_hbm.at[base + rows_per_w - 2], ssem0).wait()
        pltpu.make_async_copy(
            s64_v.at[1], out_hbm.at[base + rows_per_w - 1], ssem1).wait()

    return gather_kernel


def kernel(x, table):
    B, H = x.shape
    V = table.shape[0]
    t128 = jnp.pad(table, ((0, 0), (0, DP - D)))
    return _make_gather(B, H, V)(x.astype(jnp.int32), t128)


# speedup vs baseline: 1.0891x; 1.0891x over previous
import functools

import jax
import jax.numpy as jnp
from jax import lax
from jax.experimental import pallas as pl
from jax.experimental.pallas import tpu as pltpu
from jax.experimental.pallas import tpu_sc as plsc

D = 64
DP = 128


def _make_gather(B: int, H: int, V: int):
    info = plsc.get_sparse_core_info()
    NC, NS, L = info.num_cores, info.num_subcores, info.num_lanes
    NW = NC * NS
    assert B % NW == 0 and H % 8 == 0
    rows_per_w = B // NW
    mesh = plsc.VectorSubcoreMesh(core_axis_name="c", subcore_axis_name="s")

    @functools.partial(
        pl.kernel,
        mesh=mesh,
        out_type=jax.ShapeDtypeStruct((B, H, D), jnp.float32),
        scratch_types=[
            pltpu.VMEM((rows_per_w // 2, H), jnp.int32),
            pltpu.VMEM((2, H, DP), jnp.float32),
            pltpu.VMEM((2, H, D), jnp.float32),
            pltpu.SemaphoreType.DMA,
            pltpu.SemaphoreType.DMA,
            pltpu.SemaphoreType.DMA,
            pltpu.SemaphoreType.DMA,
        ],
        compiler_params=pltpu.CompilerParams(use_tc_tiling_on_sc=True),
    )
    def gather_kernel(x_hbm, t128_hbm, out_hbm, idx2d, rows_v, s64_v,
                      gsem, csem, ssem0, ssem1):
        wid = lax.axis_index("s") * NC + lax.axis_index("c")
        base = wid * rows_per_w
        half = rows_per_w // 2

        def stage_idx(h):
            pltpu.sync_copy(x_hbm.at[pl.ds(base + h * half, half)], idx2d)

        stage_idx(0)

        ssems = (ssem0, ssem1)

        def run_row(b_loc, b_glob, buf, wait_store):
            store_src = s64_v.at[buf]
            store_dst = out_hbm.at[base + b_glob]
            if wait_store:
                pltpu.make_async_copy(store_src, store_dst, ssems[buf]).wait()
            d1 = pltpu.async_copy(
                t128_hbm.at[idx2d.at[b_loc, pl.ds(0, 128)]],
                rows_v.at[buf].at[pl.ds(0, 128)], gsem)
            d2 = pltpu.async_copy(
                t128_hbm.at[idx2d.at[b_loc, pl.ds(128, H - 128)]],
                rows_v.at[buf].at[pl.ds(128, H - 128)], gsem)
            d1.wait()
            d2.wait()

            def compact(h, carry):
                for v in range(D // L):
                    s64_v[buf, h, pl.ds(v * L, L)] = (
                        rows_v[buf, h, pl.ds(v * L, L)])
                return carry

            lax.fori_loop(0, H, compact, 0, unroll=False)
            pltpu.async_copy(store_src, store_dst, ssems[buf])

        run_row(0, 0, 0, False)
        run_row(1, 1, 1, False)

        def body_a(p, carry):
            run_row(2 * p, 2 * p, 0, True)
            run_row(2 * p + 1, 2 * p + 1, 1, True)
            return carry

        lax.fori_loop(1, half // 2, body_a, 0, unroll=False)

        stage_idx(1)

        def body_b(p, carry):
            run_row(2 * p, half + 2 * p, 0, True)
            run_row(2 * p + 1, half + 2 * p + 1, 1, True)
            return carry

        lax.fori_loop(0, half // 2, body_b, 0, unroll=False)

        pltpu.make_async_copy(
            s64_v.at[0], out_hbm.at[base + rows_per_w - 2], ssem0).wait()
        pltpu.make_async_copy(
            s64_v.at[1], out_hbm.at[base + rows_per_w - 1], ssem1).wait()

    return gather_kernel


def kernel(x, table):
    B, H = x.shape
    V = table.shape[0]
    t128 = jnp.pad(table, ((0, 0), (0, DP - D)))
    return _make_gather(B, H, V)(x.astype(jnp.int32), t128)
